# baseline (speedup 1.0000x reference)
import jax
import jax.numpy as jnp
from jax.experimental import pallas as pl


def _matmul_body(x_ref, w_ref, o_ref):
    o_ref[...] = jnp.dot(x_ref[...], w_ref[...], preferred_element_type=jnp.float32)


def kernel(shuffled_features, mapping, weights):
    Bv, Nv = mapping.shape
    OCv, _, Cv = weights.shape
    Mv = Bv * Nv
    flat = shuffled_features.reshape(Nv, Cv)
    W2 = jnp.transpose(weights, (2, 1, 0)).reshape(Cv, Bv * OCv)
    BN = 2000
    vals = pl.pallas_call(
        _matmul_body,
        grid=(Nv // BN,),
        in_specs=[
            pl.BlockSpec((BN, Cv), lambda i: (i, 0)),
            pl.BlockSpec((Cv, Bv * OCv), lambda i: (0, 0)),
        ],
        out_specs=pl.BlockSpec((BN, Bv * OCv), lambda i: (i, 0)),
        out_shape=jax.ShapeDtypeStruct((Nv, Bv * OCv), jnp.float32),
    )(flat, W2)
    vals = vals.reshape(Mv, OCv)
    tgt = mapping.T.reshape(-1)
    safe = jnp.where(tgt < 0, Mv, tgt)
    out = jnp.zeros((Mv, OCv), jnp.float32).at[safe].set(vals, mode="drop")
    return out


# speedup vs baseline: 7.8589x; 7.8589x over previous
import functools

import jax
import jax.numpy as jnp
from jax import lax
from jax.experimental import pallas as pl
from jax.experimental.pallas import tpu as pltpu
from jax.experimental.pallas import tpu_sc as plsc

_NC = 2
_NS = 16
_NW = _NC * _NS

_SUB = 128
_NSUBBUF = 16
_SUP = _SUB * _NSUBBUF


def _matmul_body(x_ref, w_ref, o_ref):
    o_ref[...] = jnp.dot(x_ref[...], w_ref[...], preferred_element_type=jnp.float32)


def _tc_matmul(flat, W2, Nv, K, P):
    BN = 2000
    return pl.pallas_call(
        _matmul_body,
        grid=(Nv // BN,),
        in_specs=[
            pl.BlockSpec((BN, K), lambda i: (i, 0)),
            pl.BlockSpec((K, P), lambda i: (0, 0)),
        ],
        out_specs=pl.BlockSpec((BN, P), lambda i: (i, 0)),
        out_shape=jax.ShapeDtypeStruct((Nv, P), jnp.float32),
    )(flat, W2)


def _make_scatter(Mv, OCv):
    nchunk = Mv // _SUB
    nsup_full = nchunk // _NSUBBUF
    ntail = nchunk - nsup_full * _NSUBBUF
    base_n = nsup_full // _NW
    extra = nsup_full - base_n * _NW

    mesh = plsc.VectorSubcoreMesh(core_axis_name="c", subcore_axis_name="s")

    @functools.partial(
        pl.kernel,
        mesh=mesh,
        out_type=(),
        compiler_params=pltpu.CompilerParams(use_tc_tiling_on_sc=False),
        scratch_types=[
            pltpu.VMEM((_NSUBBUF, _SUB), jnp.int32),
            pltpu.VMEM((_NSUBBUF, _SUB, OCv), jnp.float32),
            pltpu.SemaphoreType.DMA,
        ],
    )
    def scatter_kernel(vals_hbm, idx_hbm, out_hbm, idx_v, rows_v, sem):
        c = lax.axis_index("c")
        s = lax.axis_index("s")
        w = s * _NC + c
        nsup_w = jnp.where(w < extra, base_n + 1, base_n)
        start_w = w * base_n + jnp.minimum(w, extra)

        def body(i, carry):
            sup = start_w + i
            pltpu.sync_copy(idx_hbm.at[pl.ds(sup * _NSUBBUF, _NSUBBUF)], idx_v)
            pltpu.sync_copy(vals_hbm.at[pl.ds(sup * _NSUBBUF, _NSUBBUF)], rows_v)
            cps = [
                pltpu.async_copy(rows_v.at[j], out_hbm.at[idx_v.at[j]], sem)
                for j in range(_NSUBBUF)
            ]
            for cp in cps:
                cp.wait()
            return carry

        lax.fori_loop(0, nsup_w, body, 0)

        if ntail:
            @pl.when(w == extra)
            def _():
                tbase = nsup_full * _NSUBBUF
                pltpu.sync_copy(idx_hbm.at[pl.ds(tbase, ntail)],
                                idx_v.at[pl.ds(0, ntail)])
                pltpu.sync_copy(vals_hbm.at[pl.ds(tbase, ntail)],
                                rows_v.at[pl.ds(0, ntail)])
                cps = [
                    pltpu.async_copy(rows_v.at[j], out_hbm.at[idx_v.at[j]], sem)
                    for j in range(ntail)
                ]
                for cp in cps:
                    cp.wait()

    return scatter_kernel


def kernel(shuffled_features, mapping, weights):
    Bv, Nv = mapping.shape
    OCv, _, Cv = weights.shape
    Mv = Bv * Nv
    flat = shuffled_features.reshape(Nv, Cv)
    W2 = jnp.transpose(weights, (2, 1, 0)).reshape(Cv, Bv * OCv)
    vals = _tc_matmul(flat, W2, Nv, Cv, Bv * OCv).reshape(Mv, OCv)

    tgt = mapping.T.reshape(-1)
    safe = jnp.maximum(tgt, 0)

    nchunk = Mv // _SUB
    idx2 = safe.reshape(nchunk, _SUB)
    vals3 = vals.reshape(nchunk, _SUB, OCv)

    out_ref = jax.new_ref(jnp.zeros((Mv, OCv), jnp.float32))
    _make_scatter(Mv, OCv)(vals3, idx2, out_ref)
    out = out_ref[...]

    hit = tgt == 0
    has = jnp.any(hit)
    r0 = jnp.argmax(hit)
    row0 = jnp.where(has, vals[r0], jnp.zeros((OCv,), jnp.float32))
    return out.at[0].set(row0)


# speedup vs baseline: 7.8618x; 1.0004x over previous
import functools

import jax
import jax.numpy as jnp
from jax import lax
from jax.experimental import pallas as pl
from jax.experimental.pallas import tpu as pltpu
from jax.experimental.pallas import tpu_sc as plsc

_NC = 2
_NS = 16
_NW = _NC * _NS

_SUB = 128
_NSUBBUF = 16
_SUP = _SUB * _NSUBBUF


def _matmul_body(x_ref, w_ref, o_ref):
    o_ref[...] = jnp.dot(x_ref[...], w_ref[...], preferred_element_type=jnp.float32)


def _tc_matmul(flat, W2, Nv, K, P):
    BN = 2000
    return pl.pallas_call(
        _matmul_body,
        grid=(Nv // BN,),
        in_specs=[
            pl.BlockSpec((BN, K), lambda i: (i, 0)),
            pl.BlockSpec((K, P), lambda i: (0, 0)),
        ],
        out_specs=pl.BlockSpec((BN, P), lambda i: (i, 0)),
        out_shape=jax.ShapeDtypeStruct((Nv, P), jnp.float32),
    )(flat, W2)


def _make_scatter(Mv, OCv):
    nchunk = Mv // _SUB
    nsup_full = nchunk // _NSUBBUF
    ntail = nchunk - nsup_full * _NSUBBUF
    base_n = nsup_full // _NW
    extra = nsup_full - base_n * _NW

    mesh = plsc.VectorSubcoreMesh(core_axis_name="c", subcore_axis_name="s")

    ntail_rows = ntail * _SUB

    scratch = [
        pltpu.VMEM((_SUP,), jnp.int32),
        pltpu.VMEM((_SUP, OCv), jnp.float32),
        pltpu.SemaphoreType.DMA,
    ]
    if ntail_rows:
        scratch += [
            pltpu.VMEM((ntail_rows,), jnp.int32),
            pltpu.VMEM((ntail_rows, OCv), jnp.float32),
        ]

    @functools.partial(
        pl.kernel,
        mesh=mesh,
        out_type=(),
        compiler_params=pltpu.CompilerParams(use_tc_tiling_on_sc=False),
        scratch_types=scratch,
    )
    def scatter_kernel(vals_hbm, idx_hbm, out_hbm, idx_v, rows_v, sem,
                       *tail_bufs):
        c = lax.axis_index("c")
        s = lax.axis_index("s")
        w = s * _NC + c
        nsup_w = jnp.where(w < extra, base_n + 1, base_n)
        start_w = w * base_n + jnp.minimum(w, extra)

        def body(i, carry):
            base = (start_w + i) * _SUP
            pltpu.sync_copy(idx_hbm.at[pl.ds(base, _SUP)], idx_v)
            pltpu.sync_copy(vals_hbm.at[pl.ds(base, _SUP)], rows_v)
            pltpu.async_copy(rows_v, out_hbm.at[idx_v], sem).wait()
            return carry

        lax.fori_loop(0, nsup_w, body, 0)

        if ntail_rows:
            idx_t, rows_t = tail_bufs

            @pl.when(w == extra)
            def _():
                tbase = nsup_full * _SUP
                pltpu.sync_copy(idx_hbm.at[pl.ds(tbase, ntail_rows)], idx_t)
                pltpu.sync_copy(vals_hbm.at[pl.ds(tbase, ntail_rows)], rows_t)
                pltpu.async_copy(rows_t, out_hbm.at[idx_t], sem).wait()

    return scatter_kernel


def kernel(shuffled_features, mapping, weights):
    Bv, Nv = mapping.shape
    OCv, _, Cv = weights.shape
    Mv = Bv * Nv
    flat = shuffled_features.reshape(Nv, Cv)
    W2 = jnp.transpose(weights, (2, 1, 0)).reshape(Cv, Bv * OCv)
    vals = _tc_matmul(flat, W2, Nv, Cv, Bv * OCv).reshape(Mv, OCv)

    tgt = mapping.T.reshape(-1)
    safe = jnp.maximum(tgt, 0)

    out_ref = jax.new_ref(jnp.zeros((Mv, OCv), jnp.float32))
    _make_scatter(Mv, OCv)(vals, safe, out_ref)
    out = out_ref[...]

    hit = tgt == 0
    has = jnp.any(hit)
    r0 = jnp.argmax(hit)
    row0 = jnp.where(has, vals[r0], jnp.zeros((OCv,), jnp.float32))
    return out.at[0].set(row0)


# speedup vs baseline: 8.9444x; 1.1377x over previous
import functools

import jax
import jax.numpy as jnp
from jax import lax
from jax.experimental import pallas as pl
from jax.experimental.pallas import tpu as pltpu
from jax.experimental.pallas import tpu_sc as plsc

_NC = 2
_NS = 16
_NW = _NC * _NS

_K = 256


def _make_matmul_body(BN, K):
    def _matmul_body(x_ref, w_ref, o_ref):
        o_ref[...] = jnp.dot(x_ref[...], w_ref[...],
                             preferred_element_type=jnp.float32)
    return _matmul_body


def _tc_matmul(flat, W2, Nv, K, P):
    BN = 2000
    return pl.pallas_call(
        _make_matmul_body(BN, K),
        grid=(Nv // BN,),
        in_specs=[
            pl.BlockSpec((BN, K), lambda i: (i, 0)),
            pl.BlockSpec((K, P), lambda i: (0, 0)),
        ],
        out_specs=pl.BlockSpec((BN, P), lambda i: (i, 0)),
        out_shape=jax.ShapeDtypeStruct((Nv, P), jnp.float32),
    )(flat, W2)


def _make_scatter(Nv, Bv, OCv):
    nfull = Nv // _K
    part = Nv - nfull * _K
    base_c = nfull // _NW
    extra = nfull - base_c * _NW

    mesh = plsc.VectorSubcoreMesh(core_axis_name="c", subcore_axis_name="s")

    scratch = [
        pltpu.VMEM((Bv, _K), jnp.int32),
        pltpu.VMEM((Bv, _K, OCv), jnp.float32),
        pltpu.SemaphoreType.DMA,
    ]
    if part:
        scratch += [
            pltpu.VMEM((Bv, part), jnp.int32),
            pltpu.VMEM((Bv, part, OCv), jnp.float32),
        ]

    @functools.partial(
        pl.kernel,
        mesh=mesh,
        out_type=(),
        compiler_params=pltpu.CompilerParams(use_tc_tiling_on_sc=False),
        scratch_types=scratch,
    )
    def scatter_kernel(vals_hbm, idx_hbm, out_hbm, idx_v, rows_v, sem,
                       *part_bufs):
        c = lax.axis_index("c")
        s = lax.axis_index("s")
        w = s * _NC + c
        n_w = jnp.where(w < extra, base_c + 1, base_c)
        start_w = w * base_c + jnp.minimum(w, extra)

        def body(i, carry):
            n0 = (start_w + i) * _K
            pltpu.sync_copy(idx_hbm.at[:, pl.ds(n0, _K)], idx_v)
            loads = [
                pltpu.async_copy(
                    vals_hbm.at[pl.ds(n0, _K), pl.ds(j * OCv, OCv)],
                    rows_v.at[j],
                    sem,
                )
                for j in range(Bv)
            ]
            for cp in loads:
                cp.wait()
            cps = [
                pltpu.async_copy(
                    rows_v.at[j],
                    out_hbm.at[idx_v.at[j]],
                    sem,
                )
                for j in range(Bv)
            ]
            for cp in cps:
                cp.wait()
            return carry

        lax.fori_loop(0, n_w, body, 0)

        if part:
            idx_p, rows_p = part_bufs

            @pl.when(w == extra)
            def _():
                n0 = nfull * _K
                pltpu.sync_copy(idx_hbm.at[:, pl.ds(n0, part)], idx_p)
                loads = [
                    pltpu.async_copy(
                        vals_hbm.at[pl.ds(n0, part), pl.ds(j * OCv, OCv)],
                        rows_p.at[j],
                        sem,
                    )
                    for j in range(Bv)
                ]
                for cp in loads:
                    cp.wait()
                cps = [
                    pltpu.async_copy(
                        rows_p.at[j],
                        out_hbm.at[idx_p.at[j]],
                        sem,
                    )
                    for j in range(Bv)
                ]
                for cp in cps:
                    cp.wait()

    return scatter_kernel


def kernel(shuffled_features, mapping, weights):
    Bv, Nv = mapping.shape
    OCv, _, Cv = weights.shape
    Mv = Bv * Nv
    flat = shuffled_features.reshape(Nv, Cv)
    W2 = jnp.transpose(weights, (2, 1, 0)).reshape(Cv, Bv * OCv)
    vals_wide = _tc_matmul(flat, W2, Nv, Cv, Bv * OCv)

    safe = jnp.maximum(mapping, 0)

    out_ref = jax.new_ref(jnp.zeros((Mv, OCv), jnp.float32))
    _make_scatter(Nv, Bv, OCv)(vals_wide, safe, out_ref)
    out = out_ref[...]

    hit = mapping == 0
    has = jnp.any(hit)
    q = jnp.argmax(hit)
    n0 = q % Nv
    j0 = q // Nv
    row0_src = lax.dynamic_slice(vals_wide, (n0, j0 * OCv), (1, OCv))
    row0 = jnp.where(has, row0_src.reshape(OCv),
                     jnp.zeros((OCv,), jnp.float32))
    return out.at[0].set(row0)
